# baseline (device time: 34678 ns/iter reference)
import jax
import jax.numpy as jnp
from jax import lax
from jax.experimental import pallas as pl
from jax.experimental.pallas import tpu as pltpu

B, S, H, Dh, Dr = 2, 256, 16, 64, 32
D = 1024
DC_SH = 64
BS = B * S
HALF = S // 2


def _dot(a, b):
    return jnp.dot(a, b, preferred_element_type=jnp.float32)


def _dot_t(a, b):
    return lax.dot_general(
        a, b, (((1,), (1,)), ((), ())), preferred_element_type=jnp.float32
    )


def kernel(x, Wdkv, Wuk, Wuv, Wq, Wqr, Wkr, Wo):
    def body(
        x_ref, wdkv_ref, wuk_ref, wuv_ref, wq_ref, wqr_ref, wkr_ref, wo_ref,
        out_ref,
        x_v, wdkv_v, wuk_v, wuv_v, wq_v, wqr_v, wkr_v, wo_v, out_v,
        wuk_send, wuv_send, wuk_rem, wuv_rem, c_buf, c_send, c_rem,
        q_buf, qr_buf, kr_buf, k_buf, v_buf, o_buf, o_buf2, ysend, yrcv,
        load_sems, store_sems, xsend_sems, xrecv_sems, ysend_sem, yrecv_sem,
    ):
        my_x = lax.axis_index("x")
        my_y = lax.axis_index("y")
        xnbr = (1 - my_x, my_y)
        ynbr = (my_x, 1 - my_y)

        loads = []
        for i, (src, dst) in enumerate([
            (wuk_ref, wuk_v), (wuv_ref, wuv_v),
            (x_ref, x_v), (wdkv_ref, wdkv_v),
            (wq_ref, wq_v), (wqr_ref, wqr_v), (wkr_ref, wkr_v),
            (wo_ref, wo_v),
        ]):
            cp = pltpu.make_async_copy(src, dst, load_sems.at[i])
            cp.start()
            loads.append(cp)

        barrier_sem = pltpu.get_barrier_semaphore()
        for nbr in (xnbr, ynbr):
            pl.semaphore_signal(
                barrier_sem, inc=1, device_id=nbr,
                device_id_type=pl.DeviceIdType.MESH,
            )
        pl.semaphore_wait(barrier_sem, 2)

        loads[0].wait()
        loads[1].wait()
        wuk_send[:] = wuk_v[:].astype(jnp.bfloat16)
        wuv_send[:] = wuv_v[:].astype(jnp.bfloat16)
        x_rdmas = []
        for i, (src, dst) in enumerate(
            [(wuk_send, wuk_rem), (wuv_send, wuv_rem)]
        ):
            r = pltpu.make_async_remote_copy(
                src_ref=src, dst_ref=dst,
                send_sem=xsend_sems.at[i], recv_sem=xrecv_sems.at[i],
                device_id=xnbr, device_id_type=pl.DeviceIdType.MESH,
            )
            r.start()
            x_rdmas.append(r)

        loads[2].wait()
        loads[3].wait()
        x2 = x_v[:].reshape(BS, D)
        c_buf[:] = _dot(x2, wdkv_v[:])
        c_send[:] = c_buf[:].astype(jnp.bfloat16)
        r = pltpu.make_async_remote_copy(
            src_ref=c_send, dst_ref=c_rem,
            send_sem=xsend_sems.at[2], recv_sem=xrecv_sems.at[2],
            device_id=xnbr, device_id_type=pl.DeviceIdType.MESH,
        )
        r.start()
        x_rdmas.append(r)

        scale = (Dh + Dr) ** -0.5
        loads[4].wait()
        q_buf[:] = _dot(x2, wq_v[:]) * scale
        loads[5].wait()
        qr_buf[:] = _dot(x2, wqr_v[:]) * scale
        loads[6].wait()
        kr_buf[:] = _dot(x2, wkr_v[:])

        for r in x_rdmas:
            r.wait()

        k_buf[:] = _dot(c_buf[:], wuk_v[:]) + _dot(c_rem[:], wuk_rem[:])
        v_buf[:] = _dot(c_buf[:], wuv_v[:]) + _dot(c_rem[:], wuv_rem[:])

        def attn(batch_base, r0, nrows, o_ref):
            rows = pl.ds(batch_base + r0, nrows)
            q_blk = q_buf[rows, :]
            qr_blk = qr_buf[rows, :]
            keys = pl.ds(batch_base, S)
            k_b = k_buf[keys, :]
            v_b = v_buf[keys, :]
            kr_b = kr_buf[keys, :]
            for h in range(H):
                s = (
                    _dot_t(q_blk[:, h * Dh:(h + 1) * Dh],
                           k_b[:, h * Dh:(h + 1) * Dh])
                    + _dot_t(qr_blk[:, h * Dr:(h + 1) * Dr], kr_b)
                )
                e = jnp.exp(s)
                o_h = _dot(e, v_b[:, h * Dh:(h + 1) * Dh])
                o_ref[0:nrows, h * Dh:(h + 1) * Dh] = o_h / jnp.sum(
                    e, axis=-1, keepdims=True
                )

        my_base = my_y * S
        other_base = (1 - my_y) * S

        def flush(b_idx, r0, nrows, sem_i):
            cp = pltpu.make_async_copy(
                out_v.at[b_idx, pl.ds(r0, nrows)],
                out_ref.at[b_idx, pl.ds(r0, nrows)],
                store_sems.at[sem_i],
            )
            cp.start()
            return cp

        attn(my_base, 0, S, o_buf)
        loads[7].wait()
        half_out = _dot(o_buf[0:HALF, :], wo_v[:])
        out_v[my_y, pl.ds(0, HALF), :] = half_out
        ysend[:] = half_out.astype(jnp.bfloat16)
        yr = pltpu.make_async_remote_copy(
            src_ref=ysend, dst_ref=yrcv,
            send_sem=ysend_sem, recv_sem=yrecv_sem,
            device_id=ynbr, device_id_type=pl.DeviceIdType.MESH,
        )
        yr.start()
        stores = [flush(my_y, 0, HALF, 0)]

        out_v[my_y, pl.ds(HALF, HALF), :] = _dot(o_buf[HALF:S, :], wo_v[:])
        stores.append(flush(my_y, HALF, HALF, 1))
        attn(other_base, HALF, HALF, o_buf2)
        out_v[1 - my_y, pl.ds(HALF, HALF), :] = _dot(o_buf2[:], wo_v[:])
        stores.append(flush(1 - my_y, HALF, HALF, 2))

        yr.wait()
        out_v[1 - my_y, pl.ds(0, HALF), :] = yrcv[:].astype(jnp.float32)
        stores.append(flush(1 - my_y, 0, HALF, 3))
        for cp in stores:
            cp.wait()

    return pl.pallas_call(
        body,
        out_shape=jax.ShapeDtypeStruct((B, S, D), jnp.float32),
        in_specs=[pl.BlockSpec(memory_space=pltpu.MemorySpace.HBM)] * 8,
        out_specs=pl.BlockSpec(memory_space=pltpu.MemorySpace.HBM),
        scratch_shapes=[
            pltpu.VMEM((B, S, D), jnp.float32),
            pltpu.VMEM((D, DC_SH), jnp.float32),
            pltpu.VMEM((DC_SH, D), jnp.float32),
            pltpu.VMEM((DC_SH, D), jnp.float32),
            pltpu.VMEM((D, D), jnp.float32),
            pltpu.VMEM((D, H * Dr), jnp.float32),
            pltpu.VMEM((D, Dr), jnp.float32),
            pltpu.VMEM((D, D), jnp.float32),
            pltpu.VMEM((B, S, D), jnp.float32),
            pltpu.VMEM((DC_SH, D), jnp.bfloat16),
            pltpu.VMEM((DC_SH, D), jnp.bfloat16),
            pltpu.VMEM((DC_SH, D), jnp.bfloat16),
            pltpu.VMEM((DC_SH, D), jnp.bfloat16),
            pltpu.VMEM((BS, DC_SH), jnp.float32),
            pltpu.VMEM((BS, DC_SH), jnp.bfloat16),
            pltpu.VMEM((BS, DC_SH), jnp.bfloat16),
            pltpu.VMEM((BS, H * Dh), jnp.float32),
            pltpu.VMEM((BS, H * Dr), jnp.float32),
            pltpu.VMEM((BS, Dr), jnp.float32),
            pltpu.VMEM((BS, H * Dh), jnp.float32),
            pltpu.VMEM((BS, H * Dh), jnp.float32),
            pltpu.VMEM((S, H * Dh), jnp.float32),
            pltpu.VMEM((HALF, H * Dh), jnp.float32),
            pltpu.VMEM((HALF, D), jnp.bfloat16),
            pltpu.VMEM((HALF, D), jnp.bfloat16),
            pltpu.SemaphoreType.DMA((8,)),
            pltpu.SemaphoreType.DMA((4,)),
            pltpu.SemaphoreType.DMA((3,)),
            pltpu.SemaphoreType.DMA((3,)),
            pltpu.SemaphoreType.DMA,
            pltpu.SemaphoreType.DMA,
        ],
        compiler_params=pltpu.CompilerParams(collective_id=0),
    )(x, Wdkv, Wuk, Wuv, Wq, Wqr, Wkr, Wo)


# device time: 32848 ns/iter; 1.0557x vs baseline; 1.0557x over previous
import functools

import jax
import jax.numpy as jnp
from jax import lax
from jax.experimental import pallas as pl
from jax.experimental.pallas import tpu as pltpu

B, S, H, Dh, Dr = 2, 256, 16, 64, 32
D = 1024
DC_SH = 64
BS = B * S
HALF = S // 2


def _dot(a, b):
    return jnp.dot(a, b, preferred_element_type=jnp.float32)


def _dot_t(a, b):
    return lax.dot_general(
        a, b, (((1,), (1,)), ((), ())), preferred_element_type=jnp.float32
    )


def kernel(x, Wdkv, Wuk, Wuv, Wq, Wqr, Wkr, Wo):
    def body(
        x_ref, wdkv_ref, wuk_ref, wuv_ref, wq_ref, wqr_ref, wkr_ref, wo_ref,
        out_ref,
        wuk_send, wuv_send, wuk_rem, wuv_rem, c_buf, c_send, c_rem,
        q_buf, qr_buf, kr_buf, k_buf, v_buf, o_buf, o_buf2, ysend, yrcv,
        ybar_sem, xsend_sems, xrecv_sems, ysend_sem, yrecv_sem,
    ):
        my_x = lax.axis_index("x")
        my_y = lax.axis_index("y")
        xnbr = (1 - my_x, my_y)
        ynbr = (my_x, 1 - my_y)

        barrier_sem = pltpu.get_barrier_semaphore()
        pl.semaphore_signal(
            barrier_sem, inc=1, device_id=xnbr,
            device_id_type=pl.DeviceIdType.MESH,
        )
        pl.semaphore_signal(
            ybar_sem, inc=1, device_id=ynbr,
            device_id_type=pl.DeviceIdType.MESH,
        )
        pl.semaphore_wait(barrier_sem, 1)

        wuk_send[:] = wuk_ref[:].astype(jnp.bfloat16)
        wuv_send[:] = wuv_ref[:].astype(jnp.bfloat16)
        x_rdmas = []
        for i, (src, dst) in enumerate(
            [(wuk_send, wuk_rem), (wuv_send, wuv_rem)]
        ):
            r = pltpu.make_async_remote_copy(
                src_ref=src, dst_ref=dst,
                send_sem=xsend_sems.at[i], recv_sem=xrecv_sems.at[i],
                device_id=xnbr, device_id_type=pl.DeviceIdType.MESH,
            )
            r.start()
            x_rdmas.append(r)

        x2 = x_ref[:].reshape(BS, D)
        c_buf[:] = _dot(x2, wdkv_ref[:])
        c_send[:] = c_buf[:].astype(jnp.bfloat16)
        r = pltpu.make_async_remote_copy(
            src_ref=c_send, dst_ref=c_rem,
            send_sem=xsend_sems.at[2], recv_sem=xrecv_sems.at[2],
            device_id=xnbr, device_id_type=pl.DeviceIdType.MESH,
        )
        r.start()
        x_rdmas.append(r)

        scale = (Dh + Dr) ** -0.5
        q_buf[:] = _dot(x2, wq_ref[:]) * scale
        qr_buf[:] = _dot(x2, wqr_ref[:]) * scale
        kr_buf[:] = _dot(x2, wkr_ref[:])

        for r in x_rdmas:
            r.wait()

        k_buf[:] = _dot(c_buf[:], wuk_ref[:]) + _dot(c_rem[:], wuk_rem[:])
        v_buf[:] = _dot(c_buf[:], wuv_ref[:]) + _dot(c_rem[:], wuv_rem[:])

        def attn(batch_base, r0, nrows, o_ref):
            rows = pl.ds(batch_base + r0, nrows)
            q_blk = q_buf[rows, :]
            qr_blk = qr_buf[rows, :]
            keys = pl.ds(batch_base, S)
            k_b = k_buf[keys, :]
            v_b = v_buf[keys, :]
            kr_b = kr_buf[keys, :]
            for h in range(H):
                s = (
                    _dot_t(q_blk[:, h * Dh:(h + 1) * Dh],
                           k_b[:, h * Dh:(h + 1) * Dh])
                    + _dot_t(qr_blk[:, h * Dr:(h + 1) * Dr], kr_b)
                )
                e = jnp.exp(s)
                o_h = _dot(e, v_b[:, h * Dh:(h + 1) * Dh])
                o_ref[0:nrows, h * Dh:(h + 1) * Dh] = o_h / jnp.sum(
                    e, axis=-1, keepdims=True
                )

        my_base = my_y * S
        other_base = (1 - my_y) * S

        attn(my_base, 0, S, o_buf)
        half_out = _dot(o_buf[0:HALF, :], wo_ref[:])
        out_ref[my_y, pl.ds(0, HALF), :] = half_out
        ysend[:] = half_out.astype(jnp.bfloat16)
        pl.semaphore_wait(ybar_sem, 1)
        yr = pltpu.make_async_remote_copy(
            src_ref=ysend, dst_ref=yrcv,
            send_sem=ysend_sem, recv_sem=yrecv_sem,
            device_id=ynbr, device_id_type=pl.DeviceIdType.MESH,
        )
        yr.start()

        out_ref[my_y, pl.ds(HALF, HALF), :] = _dot(o_buf[HALF:S, :], wo_ref[:])
        attn(other_base, HALF, HALF, o_buf2)
        out_ref[1 - my_y, pl.ds(HALF, HALF), :] = _dot(o_buf2[:], wo_ref[:])

        yr.wait()
        out_ref[1 - my_y, pl.ds(0, HALF), :] = yrcv[:].astype(jnp.float32)

    return pl.pallas_call(
        body,
        out_shape=jax.ShapeDtypeStruct((B, S, D), jnp.float32),
        in_specs=[pl.BlockSpec(memory_space=pltpu.VMEM)] * 8,
        out_specs=pl.BlockSpec(memory_space=pltpu.VMEM),
        scratch_shapes=[
            pltpu.VMEM((DC_SH, D), jnp.bfloat16),
            pltpu.VMEM((DC_SH, D), jnp.bfloat16),
            pltpu.VMEM((DC_SH, D), jnp.bfloat16),
            pltpu.VMEM((DC_SH, D), jnp.bfloat16),
            pltpu.VMEM((BS, DC_SH), jnp.float32),
            pltpu.VMEM((BS, DC_SH), jnp.bfloat16),
            pltpu.VMEM((BS, DC_SH), jnp.bfloat16),
            pltpu.VMEM((BS, H * Dh), jnp.float32),
            pltpu.VMEM((BS, H * Dr), jnp.float32),
            pltpu.VMEM((BS, Dr), jnp.float32),
            pltpu.VMEM((BS, H * Dh), jnp.float32),
            pltpu.VMEM((BS, H * Dh), jnp.float32),
            pltpu.VMEM((S, H * Dh), jnp.float32),
            pltpu.VMEM((HALF, H * Dh), jnp.float32),
            pltpu.VMEM((HALF, D), jnp.bfloat16),
            pltpu.VMEM((HALF, D), jnp.bfloat16),
            pltpu.SemaphoreType.REGULAR,
            pltpu.SemaphoreType.DMA((3,)),
            pltpu.SemaphoreType.DMA((3,)),
            pltpu.SemaphoreType.DMA,
            pltpu.SemaphoreType.DMA,
        ],
        compiler_params=pltpu.CompilerParams(collective_id=0),
    )(x, Wdkv, Wuk, Wuv, Wq, Wqr, Wkr, Wo)


# device time: 32719 ns/iter; 1.0599x vs baseline; 1.0039x over previous
import functools

import jax
import jax.numpy as jnp
from jax import lax
from jax.experimental import pallas as pl
from jax.experimental.pallas import tpu as pltpu

B, S, H, Dh, Dr = 2, 256, 16, 64, 32
D = 1024
DC_SH = 64
BS = B * S
HALF = S // 2


def _dot(a, b):
    return jnp.dot(a, b, preferred_element_type=jnp.float32)


def _dot_t(a, b):
    return lax.dot_general(
        a, b, (((1,), (1,)), ((), ())), preferred_element_type=jnp.float32
    )


def kernel(x, Wdkv, Wuk, Wuv, Wq, Wqr, Wkr, Wo):
    def body(
        x_ref, wdkv_ref, wuk_ref, wuv_ref, wq_ref, wqr_ref, wkr_ref, wo_ref,
        out_ref,
        wuk_send, wuv_send, wuk_rem, wuv_rem, c_buf, c_send, c_rem,
        q_buf, qr_buf, kr_buf, k_buf, v_buf, o_buf, o_buf2, ysend, yrcv,
        ybar_sem, xsend_sems, xrecv_sems, ysend_sem, yrecv_sem,
    ):
        my_x = lax.axis_index("x")
        my_y = lax.axis_index("y")
        xnbr = (1 - my_x, my_y)
        ynbr = (my_x, 1 - my_y)

        barrier_sem = pltpu.get_barrier_semaphore()
        pl.semaphore_signal(
            barrier_sem, inc=1, device_id=xnbr,
            device_id_type=pl.DeviceIdType.MESH,
        )
        pl.semaphore_signal(
            ybar_sem, inc=1, device_id=ynbr,
            device_id_type=pl.DeviceIdType.MESH,
        )
        wuk_send[:] = wuk_ref[:].astype(jnp.bfloat16)
        wuv_send[:] = wuv_ref[:].astype(jnp.bfloat16)
        pl.semaphore_wait(barrier_sem, 1)
        x_rdmas = []
        for i, (src, dst) in enumerate(
            [(wuk_send, wuk_rem), (wuv_send, wuv_rem)]
        ):
            r = pltpu.make_async_remote_copy(
                src_ref=src, dst_ref=dst,
                send_sem=xsend_sems.at[i], recv_sem=xrecv_sems.at[i],
                device_id=xnbr, device_id_type=pl.DeviceIdType.MESH,
            )
            r.start()
            x_rdmas.append(r)

        x2 = x_ref[:].reshape(BS, D)
        c_buf[:] = _dot(x2, wdkv_ref[:])
        c_send[:] = c_buf[:].astype(jnp.bfloat16)
        r = pltpu.make_async_remote_copy(
            src_ref=c_send, dst_ref=c_rem,
            send_sem=xsend_sems.at[2], recv_sem=xrecv_sems.at[2],
            device_id=xnbr, device_id_type=pl.DeviceIdType.MESH,
        )
        r.start()
        x_rdmas.append(r)

        scale = (Dh + Dr) ** -0.5
        q_buf[:] = (_dot(x2, wq_ref[:]) * scale).astype(jnp.bfloat16)
        qr_buf[:] = (_dot(x2, wqr_ref[:]) * scale).astype(jnp.bfloat16)
        kr_buf[:] = _dot(x2, wkr_ref[:]).astype(jnp.bfloat16)

        for r in x_rdmas:
            r.wait()

        k_buf[:] = (
            _dot(c_buf[:], wuk_ref[:]) + _dot(c_rem[:], wuk_rem[:])
        ).astype(jnp.bfloat16)
        v_buf[:] = (
            _dot(c_buf[:], wuv_ref[:]) + _dot(c_rem[:], wuv_rem[:])
        ).astype(jnp.bfloat16)

        def attn(batch_base, r0, nrows, o_ref):
            rows = pl.ds(batch_base + r0, nrows)
            q_blk = q_buf[rows, :]
            qr_blk = qr_buf[rows, :]
            keys = pl.ds(batch_base, S)
            k_b = k_buf[keys, :]
            v_b = v_buf[keys, :]
            kr_b = kr_buf[keys, :]
            for h in range(H):
                s = (
                    _dot_t(q_blk[:, h * Dh:(h + 1) * Dh],
                           k_b[:, h * Dh:(h + 1) * Dh])
                    + _dot_t(qr_blk[:, h * Dr:(h + 1) * Dr], kr_b)
                )
                e = jnp.exp(s)
                o_h = _dot(e.astype(jnp.bfloat16), v_b[:, h * Dh:(h + 1) * Dh])
                o_ref[0:nrows, h * Dh:(h + 1) * Dh] = o_h / jnp.sum(
                    e, axis=-1, keepdims=True
                )

        my_base = my_y * S
        other_base = (1 - my_y) * S

        attn(my_base, 0, S, o_buf)
        half_out = _dot(o_buf[0:HALF, :], wo_ref[:])
        out_ref[my_y, pl.ds(0, HALF), :] = half_out
        ysend[:] = half_out.astype(jnp.bfloat16)
        pl.semaphore_wait(ybar_sem, 1)
        yr = pltpu.make_async_remote_copy(
            src_ref=ysend, dst_ref=yrcv,
            send_sem=ysend_sem, recv_sem=yrecv_sem,
            device_id=ynbr, device_id_type=pl.DeviceIdType.MESH,
        )
        yr.start()

        out_ref[my_y, pl.ds(HALF, HALF), :] = _dot(o_buf[HALF:S, :], wo_ref[:])
        attn(other_base, HALF, HALF, o_buf2)
        out_ref[1 - my_y, pl.ds(HALF, HALF), :] = _dot(o_buf2[:], wo_ref[:])

        yr.wait()
        out_ref[1 - my_y, pl.ds(0, HALF), :] = yrcv[:].astype(jnp.float32)

    return pl.pallas_call(
        body,
        out_shape=jax.ShapeDtypeStruct((B, S, D), jnp.float32),
        in_specs=[pl.BlockSpec(memory_space=pltpu.VMEM)] * 8,
        out_specs=pl.BlockSpec(memory_space=pltpu.VMEM),
        scratch_shapes=[
            pltpu.VMEM((DC_SH, D), jnp.bfloat16),
            pltpu.VMEM((DC_SH, D), jnp.bfloat16),
            pltpu.VMEM((DC_SH, D), jnp.bfloat16),
            pltpu.VMEM((DC_SH, D), jnp.bfloat16),
            pltpu.VMEM((BS, DC_SH), jnp.float32),
            pltpu.VMEM((BS, DC_SH), jnp.bfloat16),
            pltpu.VMEM((BS, DC_SH), jnp.bfloat16),
            pltpu.VMEM((BS, H * Dh), jnp.bfloat16),
            pltpu.VMEM((BS, H * Dr), jnp.bfloat16),
            pltpu.VMEM((BS, Dr), jnp.bfloat16),
            pltpu.VMEM((BS, H * Dh), jnp.bfloat16),
            pltpu.VMEM((BS, H * Dh), jnp.bfloat16),
            pltpu.VMEM((S, H * Dh), jnp.float32),
            pltpu.VMEM((HALF, H * Dh), jnp.float32),
            pltpu.VMEM((HALF, D), jnp.bfloat16),
            pltpu.VMEM((HALF, D), jnp.bfloat16),
            pltpu.SemaphoreType.REGULAR,
            pltpu.SemaphoreType.DMA((3,)),
            pltpu.SemaphoreType.DMA((3,)),
            pltpu.SemaphoreType.DMA,
            pltpu.SemaphoreType.DMA,
        ],
        compiler_params=pltpu.CompilerParams(collective_id=0),
    )(x, Wdkv, Wuk, Wuv, Wq, Wqr, Wkr, Wo)
